# baseline (speedup 1.0000x reference)
import functools

import jax
import jax.numpy as jnp
from jax import lax
from jax.experimental import pallas as pl
from jax.experimental.pallas import tpu as pltpu
import jax.experimental.pallas.tpu_sc as plsc

B, N, E, S, D = 4, 10000, 160000, 2, 150
DP = 160
DH = DP // 2
PASSES = 2
H, OUT = 256, 10

NC, NT = 2, 16
EPT = E // NT
CH = 128
NCHUNK = (EPT + CH - 1) // CH
EPT_PAD = NCHUNK * CH
RPT = 632
NP = RPT * NT
LAST = N - (NT - 1) * RPT
TN = 2000


def _msg_body(h_ref, w_ref, b_ref, out_ref):
    x = h_ref[0]
    w = w_ref[0]
    out_ref[0, 0] = jnp.dot(x, w, preferred_element_type=jnp.float32) + b_ref[0]


def _msg_tc(h, w_t, b_p):
    return pl.pallas_call(
        _msg_body,
        grid=(B, S),
        in_specs=[
            pl.BlockSpec((1, N, D), lambda i, s: (i, 0, 0)),
            pl.BlockSpec((1, D, DP), lambda i, s: (s, 0, 0)),
            pl.BlockSpec((1, 1, DP), lambda i, s: (s, 0, 0)),
        ],
        out_specs=pl.BlockSpec((1, 1, N, DP), lambda i, s: (i, s, 0, 0)),
        out_shape=jax.ShapeDtypeStruct((B, S, N, DP), jnp.float32),
    )(h, w_t, b_p.reshape(S, 1, DP))


NF = S * NCHUNK


def _sc_body(msg_hbm, srcg0_hbm, srcg1_hbm, dstg_hbm, zeros_hbm, out_hbm,
             src_v, dst_v, rows_v, acc, sem0, sem1, ssem0, ssem1):
    c = lax.axis_index("c")
    t = lax.axis_index("s")
    srcs = (srcg0_hbm, srcg1_hbm)

    def gather(f, buf, sem):
        pltpu.async_copy(msg_hbm.at[src_v.at[f]], rows_v.at[buf], sem)

    def gwait(buf, sem):
        pltpu.make_async_copy(msg_hbm.at[src_v.at[0]], rows_v.at[buf], sem).wait()

    def sstart(f, buf, sem):
        pltpu.async_copy(rows_v.at[buf], acc.at[dst_v.at[f]], sem, add=True)

    def swait(buf, sem):
        pltpu.make_async_copy(rows_v.at[buf], acc.at[dst_v.at[0]], sem).wait()

    for ib in range(B // NC):
        i = c * (B // NC) + ib
        for s in range(S):
            pltpu.sync_copy(dstg_hbm.at[i, s, t],
                            dst_v.at[pl.ds(s * NCHUNK, NCHUNK)])
        for hf in range(2):
            pltpu.sync_copy(zeros_hbm, acc.at[pl.ds(t * RPT, RPT)])
            for s in range(S):
                pltpu.sync_copy(srcs[hf].at[i, s, t],
                                src_v.at[pl.ds(s * NCHUNK, NCHUNK)])
            plsc.subcore_barrier()
            gather(0, 0, sem0)
            gather(1, 1, sem1)

            @pl.loop(0, NF // 2)
            def _pair(jj):
                f0 = 2 * jj
                gwait(0, sem0)
                sstart(f0, 0, ssem0)
                gwait(1, sem1)
                sstart(f0 + 1, 1, ssem1)

                @pl.when(jj < NF // 2 - 1)
                def _():
                    swait(0, ssem0)
                    gather(f0 + 2, 0, sem0)
                    swait(1, ssem1)
                    gather(f0 + 3, 1, sem1)

                @pl.when(jj == NF // 2 - 1)
                def _():
                    swait(0, ssem0)
                    swait(1, ssem1)
            plsc.subcore_barrier()

            @pl.when(t < NT - 1)
            def _():
                pltpu.sync_copy(
                    acc.at[pl.ds(t * RPT, RPT)],
                    out_hbm.at[i, pl.ds(t * RPT, RPT), pl.ds(hf * DH, DH)])

            @pl.when(t == NT - 1)
            def _():
                pltpu.sync_copy(
                    acc.at[pl.ds((NT - 1) * RPT, LAST)],
                    out_hbm.at[i, pl.ds((NT - 1) * RPT, LAST), pl.ds(hf * DH, DH)])
            plsc.subcore_barrier()


@functools.cache
def _build_sc_scatter():
    return functools.partial(
        pl.kernel,
        out_type=jax.ShapeDtypeStruct((B, N, DP), jnp.float32),
        mesh=plsc.VectorSubcoreMesh(core_axis_name="c", subcore_axis_name="s"),
        compiler_params=pltpu.CompilerParams(use_tc_tiling_on_sc=False),
        scratch_types=[
            pltpu.VMEM((NF, CH), jnp.int32),
            pltpu.VMEM((NF, CH), jnp.int32),
            pltpu.VMEM((2, CH, DH), jnp.float32),
            pltpu.VMEM_SHARED((NP, DH), jnp.float32),
            pltpu.SemaphoreType.DMA,
            pltpu.SemaphoreType.DMA,
            pltpu.SemaphoreType.DMA,
            pltpu.SemaphoreType.DMA,
        ],
    )(_sc_body)


def _sc_scatter(msg2, src_g0, src_g1, dst_g, zeros):
    return _build_sc_scatter()(msg2, src_g0, src_g1, dst_g, zeros)


def _gru_body(inc_ref, h_ref, wi_ref, wh_ref, bi_ref, bh_ref, out_ref):
    x = inc_ref[0][:, :D]
    hh = h_ref[0]
    dot = lambda a, w: jnp.dot(a, w, preferred_element_type=jnp.float32)
    i_r = dot(x, wi_ref[0]) + bi_ref[0]
    i_z = dot(x, wi_ref[1]) + bi_ref[1]
    i_n = dot(x, wi_ref[2]) + bi_ref[2]
    h_r = dot(hh, wh_ref[0]) + bh_ref[0]
    h_z = dot(hh, wh_ref[1]) + bh_ref[1]
    h_n = dot(hh, wh_ref[2]) + bh_ref[2]
    sig = lambda u: 1.0 / (1.0 + jnp.exp(-u))
    r = sig(i_r + h_r)
    z = sig(i_z + h_z)
    n = jnp.tanh(i_n + r * h_n)
    out_ref[0] = (1.0 - z) * n + z * hh


def _gru_tc(inc, h, wi_t, wh_t, bi, bh):
    nb = B * N // TN
    return pl.pallas_call(
        _gru_body,
        grid=(nb,),
        in_specs=[
            pl.BlockSpec((1, TN, DP), lambda i: (i, 0, 0)),
            pl.BlockSpec((1, TN, D), lambda i: (i, 0, 0)),
            pl.BlockSpec((3, D, D), lambda i: (0, 0, 0)),
            pl.BlockSpec((3, D, D), lambda i: (0, 0, 0)),
            pl.BlockSpec((3, 1, D), lambda i: (0, 0, 0)),
            pl.BlockSpec((3, 1, D), lambda i: (0, 0, 0)),
        ],
        out_specs=pl.BlockSpec((1, TN, D), lambda i: (i, 0, 0)),
        out_shape=jax.ShapeDtypeStruct((nb, TN, D), jnp.float32),
    )(inc.reshape(nb, TN, DP), h.reshape(nb, TN, D),
      wi_t, wh_t, bi.reshape(3, 1, D), bh.reshape(3, 1, D)).reshape(B, N, D)


def _readout_body(h_ref, pc_ref, w1a_ref, w1b_ref, b1_ref, w2_ref, b2_ref, out_ref):
    s = jnp.sum(h_ref[0], axis=0, keepdims=True)
    v = jnp.log(s)
    v = jnp.where(v != v, 0.0, v)
    v = jnp.maximum(v, 0.0)
    x = (jnp.dot(v, w1a_ref[...], preferred_element_type=jnp.float32)
         + pc_ref[0] * w1b_ref[...] + b1_ref[...])
    x = jnp.where(x >= 0.0, x, 0.01 * x)
    out_ref[0] = (jnp.dot(x, w2_ref[...], preferred_element_type=jnp.float32)
                  + b2_ref[...])


def _readout_tc(h, pc, w1a, w1b, b1, w2, b2):
    return pl.pallas_call(
        _readout_body,
        grid=(B,),
        in_specs=[
            pl.BlockSpec((1, N, D), lambda i: (i, 0, 0)),
            pl.BlockSpec((1, 1, 1), lambda i: (i, 0, 0)),
            pl.BlockSpec((D, H), lambda i: (0, 0)),
            pl.BlockSpec((1, H), lambda i: (0, 0)),
            pl.BlockSpec((1, H), lambda i: (0, 0)),
            pl.BlockSpec((H, OUT), lambda i: (0, 0)),
            pl.BlockSpec((1, OUT), lambda i: (0, 0)),
        ],
        out_specs=pl.BlockSpec((1, 1, OUT), lambda i: (i, 0, 0)),
        out_shape=jax.ShapeDtypeStruct((B, 1, OUT), jnp.float32),
    )(h, pc.reshape(B, 1, 1), w1a, w1b, b1, w2, b2).reshape(B, OUT)


def kernel(problemClass, nodesBatch, backwards_edge_dictBatch, W_edge, b_edge,
           W_ih, W_hh, b_ih, b_hh, fc1_W, fc1_b, fc2_W, fc2_b):
    w_t = jnp.pad(jnp.transpose(W_edge, (0, 2, 1)), ((0, 0), (0, 0), (0, DP - D)))
    b_p = jnp.pad(b_edge, ((0, 0), (0, DP - D)))
    wi_t = jnp.transpose(W_ih.reshape(3, D, D), (0, 2, 1))
    wh_t = jnp.transpose(W_hh.reshape(3, D, D), (0, 2, 1))
    bi = b_ih.reshape(3, D)
    bh = b_hh.reshape(3, D)
    w1a = jnp.transpose(fc1_W[:, :D])
    w1b = fc1_W[:, D].reshape(1, H)
    b1 = fc1_b.reshape(1, H)
    w2 = jnp.transpose(fc2_W)
    b2 = fc2_b.reshape(1, OUT)

    e = backwards_edge_dictBatch
    base = ((jnp.arange(B, dtype=jnp.int32)[:, None] * S
             + jnp.arange(S, dtype=jnp.int32)[None, :]) * N)
    src2 = 2 * (e[:, :, 0, :] + base[:, :, None]).reshape(B, S, NT, EPT)
    dst = e[:, :, 1, :].reshape(B, S, NT, EPT)
    pad = EPT_PAD - EPT
    padw = ((0, 0),) * 3 + ((0, pad),)
    src_g0 = jnp.pad(src2, padw).reshape(B, S, NT, NCHUNK, CH)
    src_g1 = jnp.pad(src2 + 1, padw, constant_values=1).reshape(B, S, NT, NCHUNK, CH)
    dst_g = jnp.pad(dst, padw, constant_values=N).reshape(B, S, NT, NCHUNK, CH)
    zeros = jnp.zeros((RPT, DH), jnp.float32)

    h = nodesBatch
    for _ in range(PASSES):
        msg = _msg_tc(h, w_t, b_p)
        inc = _sc_scatter(msg.reshape(B * S * N * 2, DH),
                          src_g0, src_g1, dst_g, zeros)
        h = _gru_tc(inc, h, wi_t, wh_t, bi, bh)
    return _readout_tc(h, problemClass, w1a, w1b, b1, w2, b2)


# speedup vs baseline: 1.1738x; 1.1738x over previous
import functools

import jax
import jax.numpy as jnp
from jax import lax
from jax.experimental import pallas as pl
from jax.experimental.pallas import tpu as pltpu
import jax.experimental.pallas.tpu_sc as plsc

B, N, E, S, D = 4, 10000, 160000, 2, 150
DP = 160
DH = DP // 2
PASSES = 2
H, OUT = 256, 10

NC, NT = 2, 16
EPT = E // NT
CH = 128
NCHUNK = (EPT + CH - 1) // CH
EPT_PAD = NCHUNK * CH
RPT = 632
NP = RPT * NT
LAST = N - (NT - 1) * RPT
TN = 2000


def _msg_body(h_ref, w_ref, b_ref, out_ref):
    x = h_ref[0]
    w = w_ref[0]
    out_ref[0, 0] = jnp.dot(x, w, preferred_element_type=jnp.float32) + b_ref[0]


def _msg_tc(h, w_t, b_p):
    return pl.pallas_call(
        _msg_body,
        grid=(2, S),
        in_specs=[
            pl.BlockSpec((1, N, D), lambda i, s: (i, 0, 0)),
            pl.BlockSpec((1, D, DP), lambda i, s: (s, 0, 0)),
            pl.BlockSpec((1, 1, DP), lambda i, s: (s, 0, 0)),
        ],
        out_specs=pl.BlockSpec((1, 1, N, DP), lambda i, s: (i, s, 0, 0)),
        out_shape=jax.ShapeDtypeStruct((2, S, N, DP), jnp.float32),
    )(h, w_t, b_p.reshape(S, 1, DP))


NF = S * NCHUNK


def _sc_body(base, msg_hbm, srcg0_hbm, srcg1_hbm, dstg_hbm, zeros_hbm, out_hbm,
             src_v, dst_v, rows_v, acc, sem0, sem1):
    c = lax.axis_index("c")
    t = lax.axis_index("s")
    srcs = (srcg0_hbm, srcg1_hbm)

    def gather(f, buf, sem):
        pltpu.async_copy(msg_hbm.at[src_v.at[f]], rows_v.at[buf], sem)

    def gwait(buf, sem):
        pltpu.make_async_copy(msg_hbm.at[src_v.at[0]], rows_v.at[buf], sem).wait()

    def scatter(f, buf):
        pltpu.sync_copy(rows_v.at[buf], acc.at[dst_v.at[f]], add=True)

    if True:
        i = base + c
        for s in range(S):
            pltpu.sync_copy(dstg_hbm.at[i, s, t],
                            dst_v.at[pl.ds(s * NCHUNK, NCHUNK)])
        for hf in range(2):
            pltpu.sync_copy(zeros_hbm, acc.at[pl.ds(t * RPT, RPT)])
            for s in range(S):
                pltpu.sync_copy(srcs[hf].at[i, s, t],
                                src_v.at[pl.ds(s * NCHUNK, NCHUNK)])
            plsc.subcore_barrier()
            gather(0, 0, sem0)

            @pl.loop(0, NF // 2)
            def _pair(jj):
                f0 = 2 * jj
                gather(f0 + 1, 1, sem1)
                gwait(0, sem0)
                scatter(f0, 0)

                @pl.when(jj < NF // 2 - 1)
                def _():
                    gather(f0 + 2, 0, sem0)
                gwait(1, sem1)
                scatter(f0 + 1, 1)
            plsc.subcore_barrier()

            @pl.when(t < NT - 1)
            def _():
                pltpu.sync_copy(
                    acc.at[pl.ds(t * RPT, RPT)],
                    out_hbm.at[c, pl.ds(t * RPT, RPT), pl.ds(hf * DH, DH)])

            @pl.when(t == NT - 1)
            def _():
                pltpu.sync_copy(
                    acc.at[pl.ds((NT - 1) * RPT, LAST)],
                    out_hbm.at[c, pl.ds((NT - 1) * RPT, LAST), pl.ds(hf * DH, DH)])
            plsc.subcore_barrier()


@functools.cache
def _build_sc_scatter(base):
    return functools.partial(
        pl.kernel,
        out_type=jax.ShapeDtypeStruct((2, N, DP), jnp.float32),
        mesh=plsc.VectorSubcoreMesh(core_axis_name="c", subcore_axis_name="s"),
        compiler_params=pltpu.CompilerParams(use_tc_tiling_on_sc=False),
        scratch_types=[
            pltpu.VMEM((NF, CH), jnp.int32),
            pltpu.VMEM((NF, CH), jnp.int32),
            pltpu.VMEM((2, CH, DH), jnp.float32),
            pltpu.VMEM_SHARED((NP, DH), jnp.float32),
            pltpu.SemaphoreType.DMA,
            pltpu.SemaphoreType.DMA,
        ],
    )(functools.partial(_sc_body, base))


def _sc_scatter(msg2, src_g0, src_g1, dst_g, zeros, base):
    return _build_sc_scatter(base)(msg2, src_g0, src_g1, dst_g, zeros)


def _gru_body(inc_ref, h_ref, wi_ref, wh_ref, bi_ref, bh_ref, out_ref):
    x = inc_ref[0][:, :D]
    hh = h_ref[0]
    dot = lambda a, w: jnp.dot(a, w, preferred_element_type=jnp.float32)
    i_r = dot(x, wi_ref[0]) + bi_ref[0]
    i_z = dot(x, wi_ref[1]) + bi_ref[1]
    i_n = dot(x, wi_ref[2]) + bi_ref[2]
    h_r = dot(hh, wh_ref[0]) + bh_ref[0]
    h_z = dot(hh, wh_ref[1]) + bh_ref[1]
    h_n = dot(hh, wh_ref[2]) + bh_ref[2]
    sig = lambda u: 1.0 / (1.0 + jnp.exp(-u))
    r = sig(i_r + h_r)
    z = sig(i_z + h_z)
    n = jnp.tanh(i_n + r * h_n)
    out_ref[0] = (1.0 - z) * n + z * hh


def _gru_tc(inc, h, wi_t, wh_t, bi, bh):
    nb = 2 * N // TN
    return pl.pallas_call(
        _gru_body,
        grid=(nb,),
        in_specs=[
            pl.BlockSpec((1, TN, DP), lambda i: (i, 0, 0)),
            pl.BlockSpec((1, TN, D), lambda i: (i, 0, 0)),
            pl.BlockSpec((3, D, D), lambda i: (0, 0, 0)),
            pl.BlockSpec((3, D, D), lambda i: (0, 0, 0)),
            pl.BlockSpec((3, 1, D), lambda i: (0, 0, 0)),
            pl.BlockSpec((3, 1, D), lambda i: (0, 0, 0)),
        ],
        out_specs=pl.BlockSpec((1, TN, D), lambda i: (i, 0, 0)),
        out_shape=jax.ShapeDtypeStruct((nb, TN, D), jnp.float32),
    )(inc.reshape(nb, TN, DP), h.reshape(nb, TN, D),
      wi_t, wh_t, bi.reshape(3, 1, D), bh.reshape(3, 1, D)).reshape(2, N, D)


def _readout_body(h_ref, pc_ref, w1a_ref, w1b_ref, b1_ref, w2_ref, b2_ref, out_ref):
    s = jnp.sum(h_ref[0], axis=0, keepdims=True)
    v = jnp.log(s)
    v = jnp.where(v != v, 0.0, v)
    v = jnp.maximum(v, 0.0)
    x = (jnp.dot(v, w1a_ref[...], preferred_element_type=jnp.float32)
         + pc_ref[0] * w1b_ref[...] + b1_ref[...])
    x = jnp.where(x >= 0.0, x, 0.01 * x)
    out_ref[0] = (jnp.dot(x, w2_ref[...], preferred_element_type=jnp.float32)
                  + b2_ref[...])


def _readout_tc(h, pc, w1a, w1b, b1, w2, b2):
    return pl.pallas_call(
        _readout_body,
        grid=(2,),
        in_specs=[
            pl.BlockSpec((1, N, D), lambda i: (i, 0, 0)),
            pl.BlockSpec((1, 1, 1), lambda i: (i, 0, 0)),
            pl.BlockSpec((D, H), lambda i: (0, 0)),
            pl.BlockSpec((1, H), lambda i: (0, 0)),
            pl.BlockSpec((1, H), lambda i: (0, 0)),
            pl.BlockSpec((H, OUT), lambda i: (0, 0)),
            pl.BlockSpec((1, OUT), lambda i: (0, 0)),
        ],
        out_specs=pl.BlockSpec((1, 1, OUT), lambda i: (i, 0, 0)),
        out_shape=jax.ShapeDtypeStruct((2, 1, OUT), jnp.float32),
    )(h, pc.reshape(2, 1, 1), w1a, w1b, b1, w2, b2).reshape(2, OUT)


def kernel(problemClass, nodesBatch, backwards_edge_dictBatch, W_edge, b_edge,
           W_ih, W_hh, b_ih, b_hh, fc1_W, fc1_b, fc2_W, fc2_b):
    w_t = jnp.pad(jnp.transpose(W_edge, (0, 2, 1)), ((0, 0), (0, 0), (0, DP - D)))
    b_p = jnp.pad(b_edge, ((0, 0), (0, DP - D)))
    wi_t = jnp.transpose(W_ih.reshape(3, D, D), (0, 2, 1))
    wh_t = jnp.transpose(W_hh.reshape(3, D, D), (0, 2, 1))
    bi = b_ih.reshape(3, D)
    bh = b_hh.reshape(3, D)
    w1a = jnp.transpose(fc1_W[:, :D])
    w1b = fc1_W[:, D].reshape(1, H)
    b1 = fc1_b.reshape(1, H)
    w2 = jnp.transpose(fc2_W)
    b2 = fc2_b.reshape(1, OUT)

    e = backwards_edge_dictBatch
    base = (((jnp.arange(B, dtype=jnp.int32) % 2)[:, None] * S
             + jnp.arange(S, dtype=jnp.int32)[None, :]) * N)
    src2 = 2 * (e[:, :, 0, :] + base[:, :, None]).reshape(B, S, NT, EPT)
    dst = e[:, :, 1, :].reshape(B, S, NT, EPT)
    pad = EPT_PAD - EPT
    padw = ((0, 0),) * 3 + ((0, pad),)
    src_g0 = jnp.pad(src2, padw).reshape(B, S, NT, NCHUNK, CH)
    src_g1 = jnp.pad(src2 + 1, padw, constant_values=1).reshape(B, S, NT, NCHUNK, CH)
    dst_g = jnp.pad(dst, padw, constant_values=N).reshape(B, S, NT, NCHUNK, CH)
    zeros = jnp.zeros((RPT, DH), jnp.float32)

    hA, hB = nodesBatch[0:2], nodesBatch[2:4]
    for _ in range(PASSES):
        msgA = _msg_tc(hA, w_t, b_p)
        msgB = _msg_tc(hB, w_t, b_p)
        incA = _sc_scatter(msgA.reshape(2 * S * N * 2, DH),
                           src_g0, src_g1, dst_g, zeros, 0)
        incB = _sc_scatter(msgB.reshape(2 * S * N * 2, DH),
                           src_g0, src_g1, dst_g, zeros, 2)
        hA = _gru_tc(incA, hA, wi_t, wh_t, bi, bh)
        hB = _gru_tc(incB, hB, wi_t, wh_t, bi, bh)
    outA = _readout_tc(hA, problemClass[0:2], w1a, w1b, b1, w2, b2)
    outB = _readout_tc(hB, problemClass[2:4], w1a, w1b, b1, w2, b2)
    return jnp.concatenate([outA, outB], axis=0)
